# baseline (speedup 1.0000x reference)
import dataclasses
import functools

import jax
import jax.numpy as jnp
import numpy as np
from jax import lax
from jax.experimental import pallas as pl
from jax.experimental.pallas import tpu as pltpu
from jax.experimental.pallas import tpu_sc as plsc

N = 10000
E = 320000
D = 128
H = 8
HD = 16
C = 7

NC = 2
NS = 16
NW = NC * NS
CH = 125
NCHW = E // (CH * NW)
NPAD = 10240
RPW = NPAD // NS
BLK = 2000
NBLK = N // BLK
EPS = 1e-9

_PERM = np.empty((D,), np.int64)
for _r in range(D // 32):
    for _i in range(16):
        _PERM[32 * _r + 2 * _i] = 32 * _r + _i
        _PERM[32 * _r + 2 * _i + 1] = 32 * _r + 16 + _i


def _mm(a, b):
    return lax.dot_general(a, b, (((1,), (0,)), ((), ())),
                           precision=lax.Precision.HIGHEST,
                           preferred_element_type=jnp.float32)


def _relu(x):
    return jnp.maximum(x, 0.0)


def _leaky(x):
    return jnp.maximum(x, 0.2 * x)


def _t0_body(f_ref, ws_ref, bs_ref, wn_ref, bn_ref, as_ref, an_ref, exp_ref,
             expa_ref, wu_ref, bu_ref, wp_ref, bp_ref, wf_ref, bf_ref,
             wsf_ref, bsf_ref, x5_ref, atse_ref, spre_ref):
    f = f_ref[...]
    fs = _relu(_mm(f, ws_ref[...]) + bs_ref[...])
    ats = _leaky(_mm(fs, as_ref[...]))
    atn = _leaky(_mm(fs, an_ref[...]))
    fn = _relu(_mm(f, wn_ref[...]) + bn_ref[...])
    x5_ref[0] = fn.astype(jnp.bfloat16)
    x5_ref[1] = (_mm(atn, exp_ref[...]) * fn).astype(jnp.bfloat16)
    atse_ref[...] = _mm(ats, expa_ref[...])
    x5_ref[2] = (_mm(f, wu_ref[...]) + bu_ref[...]).astype(jnp.bfloat16)
    x5_ref[3] = (_mm(f, wp_ref[...]) + bp_ref[...]).astype(jnp.bfloat16)
    x5_ref[4] = (_mm(f, wf_ref[...]) + bf_ref[...]).astype(jnp.bfloat16)
    spre_ref[...] = _relu(_mm(f, wsf_ref[...]) + bsf_ref[...])


def _t1_body(y5_ref, atse_ref, pg_ref, pu_ref, pp_ref, pf_ref, st_ref):
    ya = y5_ref[0, 0] + y5_ref[0, 1]
    yb = y5_ref[1, 0] + y5_ref[1, 1]
    pg = atse_ref[...] * ya + yb
    pu = _relu(y5_ref[2, 0] + y5_ref[2, 1])
    pp = _relu(y5_ref[3, 0] + y5_ref[3, 1])
    pf = _relu(y5_ref[4, 0] + y5_ref[4, 1])
    pg_ref[...] = pg
    pu_ref[...] = pu
    pp_ref[...] = pp
    pf_ref[...] = pf

    @pl.when(pl.program_id(0) == 0)
    def _():
        st_ref[...] = jnp.zeros_like(st_ref)

    st = jnp.concatenate([
        jnp.sum(pg, axis=0)[None], jnp.sum(pg * pg, axis=0)[None],
        jnp.sum(pu, axis=0)[None], jnp.sum(pu * pu, axis=0)[None],
        jnp.sum(pp, axis=0)[None], jnp.sum(pp * pp, axis=0)[None],
        jnp.sum(pf, axis=0)[None], jnp.sum(pf * pf, axis=0)[None],
    ], axis=0)
    st_ref[...] = st_ref[...] + st


def _bn_apply(p, s, ssq, gamma, beta):
    m = s / N
    v = ssq / N - m * m
    inv = lax.rsqrt(v + EPS)
    return (p - m) * inv * gamma + beta


def _t1post_body(pg_ref, pu_ref, pp_ref, pf_ref, st_ref,
                 gg_ref, gb_ref, ug_ref, ub_ref, pgm_ref, pbt_ref, fg_ref, fb_ref,
                 ws_ref, bs_ref, wn_ref, bn_ref, as_ref, an_ref, exp_ref,
                 expa_ref, wu_ref, bu_ref, wp_ref, bp_ref, wf_ref, bf_ref,
                 x5_ref, atse_ref):
    st = st_ref[...]
    fg = _bn_apply(pg_ref[...], st[0], st[1], gg_ref[...], gb_ref[...])
    fu = _bn_apply(pu_ref[...], st[2], st[3], ug_ref[...], ub_ref[...])
    fp = _bn_apply(pp_ref[...], st[4], st[5], pgm_ref[...], pbt_ref[...])
    ff = _bn_apply(pf_ref[...], st[6], st[7], fg_ref[...], fb_ref[...])

    fs = _relu(_mm(fg, ws_ref[...]) + bs_ref[...])
    ats = _leaky(_mm(fs, as_ref[...]))
    atn = _leaky(_mm(fs, an_ref[...]))
    fn = _relu(_mm(fg, wn_ref[...]) + bn_ref[...])
    x5_ref[0] = fn.astype(jnp.bfloat16)
    x5_ref[1] = (_mm(atn, exp_ref[...]) * fn).astype(jnp.bfloat16)
    atse_ref[...] = _mm(ats, expa_ref[...])
    x5_ref[2] = (_mm(fu, wu_ref[...]) + bu_ref[...]).astype(jnp.bfloat16)
    x5_ref[3] = (_mm(fp, wp_ref[...]) + bp_ref[...]).astype(jnp.bfloat16)
    x5_ref[4] = (_mm(ff, wf_ref[...]) + bf_ref[...]).astype(jnp.bfloat16)


def _t2_body(y5_ref, atse_ref, spre_ref,
             pg_ref, pu_ref, pp_ref, pf_ref, st_ref):
    ya = y5_ref[0, 0] + y5_ref[0, 1]
    yb = y5_ref[1, 0] + y5_ref[1, 1]
    pg = atse_ref[...] * ya + yb
    pu = _relu(y5_ref[2, 0] + y5_ref[2, 1])
    pp = _relu(y5_ref[3, 0] + y5_ref[3, 1])
    pf = _relu(y5_ref[4, 0] + y5_ref[4, 1])
    ps = spre_ref[...]
    pg_ref[...] = pg
    pu_ref[...] = pu
    pp_ref[...] = pp
    pf_ref[...] = pf

    @pl.when(pl.program_id(0) == 0)
    def _():
        st_ref[...] = jnp.zeros_like(st_ref)

    st = jnp.concatenate([
        jnp.sum(pg, axis=0)[None], jnp.sum(pg * pg, axis=0)[None],
        jnp.sum(pu, axis=0)[None], jnp.sum(pu * pu, axis=0)[None],
        jnp.sum(pp, axis=0)[None], jnp.sum(pp * pp, axis=0)[None],
        jnp.sum(pf, axis=0)[None], jnp.sum(pf * pf, axis=0)[None],
        jnp.sum(ps, axis=0)[None], jnp.sum(ps * ps, axis=0)[None],
    ], axis=0)
    st_ref[...] = st_ref[...] + st


def _t2post_body(pg_ref, pu_ref, pp_ref, pf_ref, ps_ref, st_ref,
                 gg_ref, gb_ref, ug_ref, ub_ref, pgm_ref, pbt_ref, fg_ref, fb_ref,
                 sg_ref, sb_ref,
                 wcg_ref, wcu_ref, wcp_ref, wcf_ref, wcs_ref, cb_ref, o_ref):
    st = st_ref[...]
    og = _bn_apply(pg_ref[...], st[0], st[1], gg_ref[...], gb_ref[...])
    ou = _bn_apply(pu_ref[...], st[2], st[3], ug_ref[...], ub_ref[...])
    op = _bn_apply(pp_ref[...], st[4], st[5], pgm_ref[...], pbt_ref[...])
    of = _bn_apply(pf_ref[...], st[6], st[7], fg_ref[...], fb_ref[...])
    os_ = _bn_apply(ps_ref[...], st[8], st[9], sg_ref[...], sb_ref[...])
    o_ref[...] = (_mm(og, wcg_ref[...]) + _mm(ou, wcu_ref[...])
                  + _mm(op, wcp_ref[...]) + _mm(of, wcf_ref[...])
                  + _mm(os_, wcs_ref[...]) + cb_ref[...])


def _sc_spmm(x5, rows5, cols5, vals5):
    mesh = plsc.VectorSubcoreMesh(core_axis_name="c", subcore_axis_name="s",
                                  num_cores=NC, num_subcores=NS)
    out_type = jax.ShapeDtypeStruct((5, NC, NPAD, D), jnp.float32)
    qtr = NCHW // 5
    scratch = [
        pltpu.VMEM_SHARED((NPAD, D), jnp.float32),
        pltpu.VMEM((qtr, CH), jnp.int32),
        pltpu.VMEM((qtr, CH), jnp.int32),
        pltpu.VMEM((qtr, CH), jnp.float32),
        pltpu.VMEM((CH, D // 2), jnp.int32),
        pltpu.VMEM((CH, D // 2), jnp.int32),
        pltpu.VMEM((CH, D), jnp.float32),
        pltpu.VMEM((16, D), jnp.float32),
        pltpu.SemaphoreType.DMA,
        pltpu.SemaphoreType.DMA,
    ]

    cp = pltpu.CompilerParams(use_tc_tiling_on_sc=False)
    if "needs_layout_passes" in pltpu.CompilerParams.__dataclass_fields__:
        cp = dataclasses.replace(cp, needs_layout_passes=False)

    @functools.partial(pl.kernel, mesh=mesh, out_type=out_type,
                       scratch_types=scratch, compiler_params=cp)
    def k(x_h, r_h, c_h, v_h, o_h, acc, rows_v, cols_v, vals_v, GA, GB, G32,
          Z, sga, sgb):
        cid = lax.axis_index("c")
        sid = lax.axis_index("s")
        wid = sid * NC + cid

        @pl.loop(0, 16)
        def _(e):
            for r in range(D // 16):
                Z[e, pl.ds(r * 16, 16)] = jnp.zeros((16,), jnp.float32)

        def scale(G, ci):
            for g in range((CH + 15) // 16):
                base = g * 16 if (g + 1) * 16 <= CH else CH - 16
                lo = 0 if (g + 1) * 16 <= CH else g * 16 - base
                vv = vals_v[ci, pl.ds(base, 16)]
                for j in range(lo, 16):
                    val = vv[j]
                    e = base + j
                    for r in range(D // 32):
                        vb = plsc.bitcast(G[e, pl.ds(r * 16, 16)],
                                          jnp.bfloat16)
                        a, b = plsc.unpack(
                            vb, format=plsc.PackFormat.INTERLEAVED)
                        G32[e, pl.ds(r * 32, 16)] = a * val
                        G32[e, pl.ds(r * 32 + 16, 16)] = b * val

        @pl.loop(0, 5)
        def _(p):
            @pl.loop(0, RPW // 16)
            def _(kk):
                pltpu.sync_copy(Z, acc.at[pl.ds(sid * RPW + kk * 16, 16)])
            plsc.subcore_barrier()

            @pl.loop(0, NCHW // qtr)
            def _(qp):
                qbase = wid * NCHW + qp * qtr
                pltpu.sync_copy(r_h.at[p, pl.ds(qbase, qtr)], rows_v)
                pltpu.sync_copy(c_h.at[p, pl.ds(qbase, qtr)], cols_v)
                pltpu.sync_copy(v_h.at[p, pl.ds(qbase, qtr)], vals_v)

                pltpu.async_copy(x_h.at[p].at[cols_v.at[0]], GA, sga)

                @pl.loop(0, qtr // 2)
                def _(kk):
                    ca = 2 * kk
                    cb = 2 * kk + 1
                    pltpu.make_async_copy(x_h.at[p].at[cols_v.at[ca]], GA,
                                          sga).wait()
                    pltpu.async_copy(x_h.at[p].at[cols_v.at[cb]], GB, sgb)
                    scale(GA, ca)
                    pltpu.sync_copy(G32, acc.at[rows_v.at[ca]], add=True)
                    pltpu.make_async_copy(x_h.at[p].at[cols_v.at[cb]], GB,
                                          sgb).wait()

                    @pl.when(cb + 1 < qtr)
                    def _():
                        pltpu.async_copy(x_h.at[p].at[cols_v.at[cb + 1]],
                                         GA, sga)

                    scale(GB, cb)
                    pltpu.sync_copy(G32, acc.at[rows_v.at[cb]], add=True)

            plsc.subcore_barrier()
            pltpu.sync_copy(acc.at[pl.ds(sid * RPW, RPW)],
                            o_h.at[p, cid, pl.ds(sid * RPW, RPW)])
            plsc.subcore_barrier()

    return k(x5, rows5, cols5, vals5)


def _blockdiag(a):
    return (a[:, :, None] * jnp.eye(H, dtype=a.dtype)[:, None, :]).reshape(D, H)


def _cat_w(w):
    return w.transpose(1, 0, 2).reshape(D, H * HD)


def kernel(f_in, edge_index0, adj_val0, edge_index1, adj_val1, edge_index2,
           adj_val2, edge_index3, adj_val3, gat_Wself, gat_bself, gat_Wneigh,
           gat_bneigh, gat_aself, gat_aneigh, gat_gamma, gat_beta, uttr_W,
           uttr_b, uttr_gamma, uttr_beta, past_W, past_b, past_gamma,
           past_beta, futr_W, futr_b, futr_gamma, futr_beta, self_W, self_b,
           self_gamma, self_beta, cls_W, cls_b):
    exp_mat = jnp.repeat(jnp.eye(H, dtype=jnp.float32), HD, axis=1)
    perm = jnp.asarray(_PERM)
    pc = lambda w: w[:, perm]
    pb = lambda b: b[perm]

    eis = [edge_index0, edge_index0, edge_index1, edge_index2, edge_index3]
    avs = [adj_val0, adj_val0, adj_val1, adj_val2, adj_val3]
    rows5 = jnp.stack([ei[0].reshape(E // CH, CH) for ei in eis])
    cols5 = jnp.stack([ei[1].reshape(E // CH, CH) for ei in eis])
    vals5 = jnp.stack([av.reshape(E // CH, CH) for av in avs])

    full = lambda s: pl.BlockSpec(s, lambda i: tuple(0 for _ in s))
    rblk = pl.BlockSpec((BLK, D), lambda i: (i, 0))
    x5blk = pl.BlockSpec((5, BLK, D), lambda i: (0, i, 0))
    y5blk = pl.BlockSpec((5, NC, BLK, D), lambda i: (0, 0, i, 0))
    w128 = full((D, D))
    b128 = full((D,))
    a8 = full((D, H))
    e8 = full((H, D))

    bd = jax.ShapeDtypeStruct((N, D), jnp.float32)

    t0_out = pl.pallas_call(
        _t0_body,
        grid=(NBLK,),
        in_specs=[rblk] + [w128, b128, w128, b128, a8, a8, e8, e8]
        + [w128, b128, w128, b128, w128, b128, w128, b128],
        out_specs=[x5blk, rblk, rblk],
        out_shape=[jax.ShapeDtypeStruct((5, N, D), jnp.bfloat16), bd, bd],
    )(f_in, _cat_w(gat_Wself[0]), gat_bself[0].reshape(D),
      pc(_cat_w(gat_Wneigh[0])), pb(gat_bneigh[0].reshape(D)),
      _blockdiag(gat_aself[0, :, :, 0]), _blockdiag(gat_aneigh[0, :, :, 0]),
      pc(exp_mat), exp_mat, pc(uttr_W[0]), pb(uttr_b[0]),
      pc(past_W[0]), pb(past_b[0]), pc(futr_W[0]), pb(futr_b[0]),
      self_W, self_b)
    x5, atse0, spre = t0_out

    x5i = lax.bitcast_convert_type(x5.reshape(5, N, D // 2, 2), jnp.int32)
    y5 = _sc_spmm(x5i, rows5, cols5, vals5)[:, :, :N]

    pg1, pu1, pp1, pf1, st1 = pl.pallas_call(
        _t1_body,
        grid=(NBLK,),
        in_specs=[y5blk, rblk],
        out_specs=[rblk] * 4 + [full((8, D))],
        out_shape=[bd] * 4 + [jax.ShapeDtypeStruct((8, D), jnp.float32)],
    )(y5, atse0)

    t1p_out = pl.pallas_call(
        _t1post_body,
        grid=(NBLK,),
        in_specs=[rblk] * 4 + [full((8, D))] + [b128] * 8
        + [w128, b128, w128, b128, a8, a8, e8, e8]
        + [w128, b128, w128, b128, w128, b128],
        out_specs=[x5blk, rblk],
        out_shape=[jax.ShapeDtypeStruct((5, N, D), jnp.bfloat16), bd],
    )(pg1, pu1, pp1, pf1, st1,
      gat_gamma[0], gat_beta[0], uttr_gamma[0], uttr_beta[0],
      past_gamma[0], past_beta[0], futr_gamma[0], futr_beta[0],
      _cat_w(gat_Wself[1]), gat_bself[1].reshape(D),
      pc(_cat_w(gat_Wneigh[1])), pb(gat_bneigh[1].reshape(D)),
      _blockdiag(gat_aself[1, :, :, 0]), _blockdiag(gat_aneigh[1, :, :, 0]),
      pc(exp_mat), exp_mat, pc(uttr_W[1]), pb(uttr_b[1]),
      pc(past_W[1]), pb(past_b[1]), pc(futr_W[1]), pb(futr_b[1]))
    x5_2, atse1 = t1p_out

    x5i_2 = lax.bitcast_convert_type(x5_2.reshape(5, N, D // 2, 2),
                                     jnp.int32)
    y5_2 = _sc_spmm(x5i_2, rows5, cols5, vals5)[:, :, :N]

    pg2, pu2, pp2, pf2, st2 = pl.pallas_call(
        _t2_body,
        grid=(NBLK,),
        in_specs=[y5blk, rblk, rblk],
        out_specs=[rblk] * 4 + [full((10, D))],
        out_shape=[bd] * 4 + [jax.ShapeDtypeStruct((10, D), jnp.float32)],
    )(y5_2, atse1, spre)

    out = pl.pallas_call(
        _t2post_body,
        grid=(NBLK,),
        in_specs=[rblk] * 5 + [full((10, D))] + [b128] * 10
        + [full((D, C))] * 5 + [full((C,))],
        out_specs=pl.BlockSpec((BLK, C), lambda i: (i, 0)),
        out_shape=jax.ShapeDtypeStruct((N, C), jnp.float32),
    )(pg2, pu2, pp2, pf2, spre, st2,
      gat_gamma[1], gat_beta[1], uttr_gamma[1], uttr_beta[1],
      past_gamma[1], past_beta[1], futr_gamma[1], futr_beta[1],
      self_gamma, self_beta,
      cls_W[0:D], cls_W[D:2 * D], cls_W[2 * D:3 * D], cls_W[3 * D:4 * D],
      cls_W[4 * D:5 * D], cls_b)
    return out


# speedup vs baseline: 1.2064x; 1.2064x over previous
import functools

import jax
import jax.numpy as jnp
from jax import lax
from jax.experimental import pallas as pl
from jax.experimental.pallas import tpu as pltpu
from jax.experimental.pallas import tpu_sc as plsc

N = 10000
E = 320000
D = 128
H = 8
HD = 16
C = 7

NC = 2
NS = 16
NW = NC * NS
CH = 125
NCHW = E // (CH * NW)
NPAD = 10240
RPW = NPAD // NS
BLK = 2000
NBLK = N // BLK
EPS = 1e-9


def _mm(a, b):
    return lax.dot_general(a, b, (((1,), (0,)), ((), ())),
                           precision=lax.Precision.HIGHEST,
                           preferred_element_type=jnp.float32)


def _relu(x):
    return jnp.maximum(x, 0.0)


def _leaky(x):
    return jnp.maximum(x, 0.2 * x)


def _t0_body(f_ref, ws_ref, bs_ref, wn_ref, bn_ref, as_ref, an_ref, exp_ref,
             wu_ref, bu_ref, wp_ref, bp_ref, wf_ref, bf_ref, wsf_ref, bsf_ref,
             x5_ref, atse_ref, spre_ref):
    f = f_ref[...]
    fs = _relu(_mm(f, ws_ref[...]) + bs_ref[...])
    ats = _leaky(_mm(fs, as_ref[...]))
    atn = _leaky(_mm(fs, an_ref[...]))
    fn = _relu(_mm(f, wn_ref[...]) + bn_ref[...])
    x5_ref[0] = fn
    x5_ref[1] = _mm(atn, exp_ref[...]) * fn
    atse_ref[...] = _mm(ats, exp_ref[...])
    x5_ref[2] = _mm(f, wu_ref[...]) + bu_ref[...]
    x5_ref[3] = _mm(f, wp_ref[...]) + bp_ref[...]
    x5_ref[4] = _mm(f, wf_ref[...]) + bf_ref[...]
    spre_ref[...] = _relu(_mm(f, wsf_ref[...]) + bsf_ref[...])


def _t1_body(y5_ref, atse_ref, pg_ref, pu_ref, pp_ref, pf_ref, st_ref):
    ya = y5_ref[0, 0] + y5_ref[0, 1]
    yb = y5_ref[1, 0] + y5_ref[1, 1]
    pg = atse_ref[...] * ya + yb
    pu = _relu(y5_ref[2, 0] + y5_ref[2, 1])
    pp = _relu(y5_ref[3, 0] + y5_ref[3, 1])
    pf = _relu(y5_ref[4, 0] + y5_ref[4, 1])
    pg_ref[...] = pg
    pu_ref[...] = pu
    pp_ref[...] = pp
    pf_ref[...] = pf

    @pl.when(pl.program_id(0) == 0)
    def _():
        st_ref[...] = jnp.zeros_like(st_ref)

    st = jnp.concatenate([
        jnp.sum(pg, axis=0)[None], jnp.sum(pg * pg, axis=0)[None],
        jnp.sum(pu, axis=0)[None], jnp.sum(pu * pu, axis=0)[None],
        jnp.sum(pp, axis=0)[None], jnp.sum(pp * pp, axis=0)[None],
        jnp.sum(pf, axis=0)[None], jnp.sum(pf * pf, axis=0)[None],
    ], axis=0)
    st_ref[...] = st_ref[...] + st


def _bn_apply(p, s, ssq, gamma, beta):
    m = s / N
    v = ssq / N - m * m
    inv = lax.rsqrt(v + EPS)
    return (p - m) * inv * gamma + beta


def _t1post_body(pg_ref, pu_ref, pp_ref, pf_ref, st_ref,
                 gg_ref, gb_ref, ug_ref, ub_ref, pgm_ref, pbt_ref, fg_ref, fb_ref,
                 ws_ref, bs_ref, wn_ref, bn_ref, as_ref, an_ref, exp_ref,
                 wu_ref, bu_ref, wp_ref, bp_ref, wf_ref, bf_ref,
                 x5_ref, atse_ref):
    st = st_ref[...]
    fg = _bn_apply(pg_ref[...], st[0], st[1], gg_ref[...], gb_ref[...])
    fu = _bn_apply(pu_ref[...], st[2], st[3], ug_ref[...], ub_ref[...])
    fp = _bn_apply(pp_ref[...], st[4], st[5], pgm_ref[...], pbt_ref[...])
    ff = _bn_apply(pf_ref[...], st[6], st[7], fg_ref[...], fb_ref[...])

    fs = _relu(_mm(fg, ws_ref[...]) + bs_ref[...])
    ats = _leaky(_mm(fs, as_ref[...]))
    atn = _leaky(_mm(fs, an_ref[...]))
    fn = _relu(_mm(fg, wn_ref[...]) + bn_ref[...])
    x5_ref[0] = fn
    x5_ref[1] = _mm(atn, exp_ref[...]) * fn
    atse_ref[...] = _mm(ats, exp_ref[...])
    x5_ref[2] = _mm(fu, wu_ref[...]) + bu_ref[...]
    x5_ref[3] = _mm(fp, wp_ref[...]) + bp_ref[...]
    x5_ref[4] = _mm(ff, wf_ref[...]) + bf_ref[...]


def _t2_body(y5_ref, atse_ref, spre_ref,
             pg_ref, pu_ref, pp_ref, pf_ref, st_ref):
    ya = y5_ref[0, 0] + y5_ref[0, 1]
    yb = y5_ref[1, 0] + y5_ref[1, 1]
    pg = atse_ref[...] * ya + yb
    pu = _relu(y5_ref[2, 0] + y5_ref[2, 1])
    pp = _relu(y5_ref[3, 0] + y5_ref[3, 1])
    pf = _relu(y5_ref[4, 0] + y5_ref[4, 1])
    ps = spre_ref[...]
    pg_ref[...] = pg
    pu_ref[...] = pu
    pp_ref[...] = pp
    pf_ref[...] = pf

    @pl.when(pl.program_id(0) == 0)
    def _():
        st_ref[...] = jnp.zeros_like(st_ref)

    st = jnp.concatenate([
        jnp.sum(pg, axis=0)[None], jnp.sum(pg * pg, axis=0)[None],
        jnp.sum(pu, axis=0)[None], jnp.sum(pu * pu, axis=0)[None],
        jnp.sum(pp, axis=0)[None], jnp.sum(pp * pp, axis=0)[None],
        jnp.sum(pf, axis=0)[None], jnp.sum(pf * pf, axis=0)[None],
        jnp.sum(ps, axis=0)[None], jnp.sum(ps * ps, axis=0)[None],
    ], axis=0)
    st_ref[...] = st_ref[...] + st


def _t2post_body(pg_ref, pu_ref, pp_ref, pf_ref, ps_ref, st_ref,
                 gg_ref, gb_ref, ug_ref, ub_ref, pgm_ref, pbt_ref, fg_ref, fb_ref,
                 sg_ref, sb_ref,
                 wcg_ref, wcu_ref, wcp_ref, wcf_ref, wcs_ref, cb_ref, o_ref):
    st = st_ref[...]
    og = _bn_apply(pg_ref[...], st[0], st[1], gg_ref[...], gb_ref[...])
    ou = _bn_apply(pu_ref[...], st[2], st[3], ug_ref[...], ub_ref[...])
    op = _bn_apply(pp_ref[...], st[4], st[5], pgm_ref[...], pbt_ref[...])
    of = _bn_apply(pf_ref[...], st[6], st[7], fg_ref[...], fb_ref[...])
    os_ = _bn_apply(ps_ref[...], st[8], st[9], sg_ref[...], sb_ref[...])
    o_ref[...] = (_mm(og, wcg_ref[...]) + _mm(ou, wcu_ref[...])
                  + _mm(op, wcp_ref[...]) + _mm(of, wcf_ref[...])
                  + _mm(os_, wcs_ref[...]) + cb_ref[...])


def _sc_spmm(x5, rows5, cols5, vals5):
    mesh = plsc.VectorSubcoreMesh(core_axis_name="c", subcore_axis_name="s",
                                  num_cores=NC, num_subcores=NS)
    out_type = jax.ShapeDtypeStruct((5, NC, NPAD, D), jnp.float32)
    qtr = NCHW // 5
    scratch = [
        pltpu.VMEM_SHARED((NPAD, D), jnp.float32),
        pltpu.VMEM((qtr, CH), jnp.int32),
        pltpu.VMEM((qtr, CH), jnp.int32),
        pltpu.VMEM((qtr, CH), jnp.float32),
        pltpu.VMEM((CH, D), jnp.float32),
        pltpu.VMEM((CH, D), jnp.float32),
        pltpu.VMEM((16, D), jnp.float32),
        pltpu.SemaphoreType.DMA,
        pltpu.SemaphoreType.DMA,
        pltpu.SemaphoreType.DMA,
        pltpu.SemaphoreType.DMA,
    ]

    @functools.partial(pl.kernel, mesh=mesh, out_type=out_type,
                       scratch_types=scratch)
    def k(x_h, r_h, c_h, v_h, o_h, acc, rows_v, cols_v, vals_v, GA, GB, Z,
          sga, sgb, ssa, ssb):
        cid = lax.axis_index("c")
        sid = lax.axis_index("s")
        wid = sid * NC + cid

        @pl.loop(0, 16)
        def _(e):
            for r in range(D // 16):
                Z[e, pl.ds(r * 16, 16)] = jnp.zeros((16,), jnp.float32)

        def scale(G, ci):
            for g in range((CH + 15) // 16):
                base = g * 16 if (g + 1) * 16 <= CH else CH - 16
                lo = 0 if (g + 1) * 16 <= CH else g * 16 - base
                vv = vals_v[ci, pl.ds(base, 16)]
                for j in range(lo, 16):
                    val = vv[j]
                    for r in range(D // 16):
                        sl = pl.ds(r * 16, 16)
                        G[base + j, sl] = G[base + j, sl] * val

        @pl.loop(0, 5)
        def _(p):
            @pl.loop(0, RPW // 16)
            def _(kk):
                pltpu.sync_copy(Z, acc.at[pl.ds(sid * RPW + kk * 16, 16)])
            plsc.subcore_barrier()

            @pl.loop(0, NCHW // qtr)
            def _(qp):
                qbase = wid * NCHW + qp * qtr
                pltpu.sync_copy(r_h.at[p, pl.ds(qbase, qtr)], rows_v)
                pltpu.sync_copy(c_h.at[p, pl.ds(qbase, qtr)], cols_v)
                pltpu.sync_copy(v_h.at[p, pl.ds(qbase, qtr)], vals_v)

                HH = 64

                def gstart(ci, G, sem):
                    pltpu.async_copy(
                        x_h.at[p].at[cols_v.at[ci, pl.ds(0, HH)]],
                        G.at[pl.ds(0, HH)], sem)
                    pltpu.async_copy(
                        x_h.at[p].at[cols_v.at[ci, pl.ds(HH, CH - HH)]],
                        G.at[pl.ds(HH, CH - HH)], sem)

                def gwait(ci, G, sem):
                    pltpu.make_async_copy(
                        x_h.at[p].at[cols_v.at[ci, pl.ds(0, HH)]],
                        G.at[pl.ds(0, HH)], sem).wait()
                    pltpu.make_async_copy(
                        x_h.at[p].at[cols_v.at[ci, pl.ds(HH, CH - HH)]],
                        G.at[pl.ds(HH, CH - HH)], sem).wait()

                gstart(0, GA, sga)

                @pl.loop(0, qtr // 2)
                def _(kk):
                    ca = 2 * kk
                    cb = 2 * kk + 1
                    gwait(ca, GA, sga)
                    gstart(cb, GB, sgb)
                    scale(GA, ca)
                    pltpu.sync_copy(GA, acc.at[rows_v.at[ca]], add=True)
                    gwait(cb, GB, sgb)

                    @pl.when(cb + 1 < qtr)
                    def _():
                        gstart(cb + 1, GA, sga)

                    scale(GB, cb)
                    pltpu.sync_copy(GB, acc.at[rows_v.at[cb]], add=True)

            plsc.subcore_barrier()
            pltpu.sync_copy(acc.at[pl.ds(sid * RPW, RPW)],
                            o_h.at[p, cid, pl.ds(sid * RPW, RPW)])
            plsc.subcore_barrier()

    return k(x5, rows5, cols5, vals5)


def _blockdiag(a):
    return (a[:, :, None] * jnp.eye(H, dtype=a.dtype)[:, None, :]).reshape(D, H)


def _cat_w(w):
    return w.transpose(1, 0, 2).reshape(D, H * HD)


def kernel(f_in, edge_index0, adj_val0, edge_index1, adj_val1, edge_index2,
           adj_val2, edge_index3, adj_val3, gat_Wself, gat_bself, gat_Wneigh,
           gat_bneigh, gat_aself, gat_aneigh, gat_gamma, gat_beta, uttr_W,
           uttr_b, uttr_gamma, uttr_beta, past_W, past_b, past_gamma,
           past_beta, futr_W, futr_b, futr_gamma, futr_beta, self_W, self_b,
           self_gamma, self_beta, cls_W, cls_b):
    exp_mat = jnp.repeat(jnp.eye(H, dtype=jnp.float32), HD, axis=1)

    eis = [edge_index0, edge_index0, edge_index1, edge_index2, edge_index3]
    avs = [adj_val0, adj_val0, adj_val1, adj_val2, adj_val3]
    rows5 = jnp.stack([ei[0].reshape(E // CH, CH) for ei in eis])
    cols5 = jnp.stack([ei[1].reshape(E // CH, CH) for ei in eis])
    vals5 = jnp.stack([av.reshape(E // CH, CH) for av in avs])

    full = lambda s: pl.BlockSpec(s, lambda i: tuple(0 for _ in s))
    rblk = pl.BlockSpec((BLK, D), lambda i: (i, 0))
    x5blk = pl.BlockSpec((5, BLK, D), lambda i: (0, i, 0))
    y5blk = pl.BlockSpec((5, NC, BLK, D), lambda i: (0, 0, i, 0))
    w128 = full((D, D))
    b128 = full((D,))
    a8 = full((D, H))
    e8 = full((H, D))

    bd = jax.ShapeDtypeStruct((N, D), jnp.float32)

    t0_out = pl.pallas_call(
        _t0_body,
        grid=(NBLK,),
        in_specs=[rblk] + [w128, b128, w128, b128, a8, a8, e8]
        + [w128, b128, w128, b128, w128, b128, w128, b128],
        out_specs=[x5blk, rblk, rblk],
        out_shape=[jax.ShapeDtypeStruct((5, N, D), jnp.float32), bd, bd],
    )(f_in, _cat_w(gat_Wself[0]), gat_bself[0].reshape(D),
      _cat_w(gat_Wneigh[0]), gat_bneigh[0].reshape(D),
      _blockdiag(gat_aself[0, :, :, 0]), _blockdiag(gat_aneigh[0, :, :, 0]),
      exp_mat, uttr_W[0], uttr_b[0], past_W[0], past_b[0],
      futr_W[0], futr_b[0], self_W, self_b)
    x5, atse0, spre = t0_out

    y5 = _sc_spmm(x5, rows5, cols5, vals5)[:, :, :N]

    pg1, pu1, pp1, pf1, st1 = pl.pallas_call(
        _t1_body,
        grid=(NBLK,),
        in_specs=[y5blk, rblk],
        out_specs=[rblk] * 4 + [full((8, D))],
        out_shape=[bd] * 4 + [jax.ShapeDtypeStruct((8, D), jnp.float32)],
    )(y5, atse0)

    t1p_out = pl.pallas_call(
        _t1post_body,
        grid=(NBLK,),
        in_specs=[rblk] * 4 + [full((8, D))] + [b128] * 8
        + [w128, b128, w128, b128, a8, a8, e8]
        + [w128, b128, w128, b128, w128, b128],
        out_specs=[x5blk, rblk],
        out_shape=[jax.ShapeDtypeStruct((5, N, D), jnp.float32), bd],
    )(pg1, pu1, pp1, pf1, st1,
      gat_gamma[0], gat_beta[0], uttr_gamma[0], uttr_beta[0],
      past_gamma[0], past_beta[0], futr_gamma[0], futr_beta[0],
      _cat_w(gat_Wself[1]), gat_bself[1].reshape(D),
      _cat_w(gat_Wneigh[1]), gat_bneigh[1].reshape(D),
      _blockdiag(gat_aself[1, :, :, 0]), _blockdiag(gat_aneigh[1, :, :, 0]),
      exp_mat, uttr_W[1], uttr_b[1], past_W[1], past_b[1], futr_W[1], futr_b[1])
    x5_2, atse1 = t1p_out

    y5_2 = _sc_spmm(x5_2, rows5, cols5, vals5)[:, :, :N]

    pg2, pu2, pp2, pf2, st2 = pl.pallas_call(
        _t2_body,
        grid=(NBLK,),
        in_specs=[y5blk, rblk, rblk],
        out_specs=[rblk] * 4 + [full((10, D))],
        out_shape=[bd] * 4 + [jax.ShapeDtypeStruct((10, D), jnp.float32)],
    )(y5_2, atse1, spre)

    out = pl.pallas_call(
        _t2post_body,
        grid=(NBLK,),
        in_specs=[rblk] * 5 + [full((10, D))] + [b128] * 10
        + [full((D, C))] * 5 + [full((C,))],
        out_specs=pl.BlockSpec((BLK, C), lambda i: (i, 0)),
        out_shape=jax.ShapeDtypeStruct((N, C), jnp.float32),
    )(pg2, pu2, pp2, pf2, spre, st2,
      gat_gamma[1], gat_beta[1], uttr_gamma[1], uttr_beta[1],
      past_gamma[1], past_beta[1], futr_gamma[1], futr_beta[1],
      self_gamma, self_beta,
      cls_W[0:D], cls_W[D:2 * D], cls_W[2 * D:3 * D], cls_W[3 * D:4 * D],
      cls_W[4 * D:5 * D], cls_b)
    return out


# speedup vs baseline: 1.2488x; 1.0351x over previous
import functools

import jax
import jax.numpy as jnp
from jax import lax
from jax.experimental import pallas as pl
from jax.experimental.pallas import tpu as pltpu
from jax.experimental.pallas import tpu_sc as plsc

N = 10000
E = 320000
D = 128
H = 8
HD = 16
C = 7

NC = 2
NS = 16
NW = NC * NS
CH = 125
NCHW = E // (CH * NW)
NPAD = 10240
RPW = NPAD // NS
BLK = 1280
NBLK = NPAD // BLK
EPS = 1e-9


def _mm(a, b):
    return lax.dot_general(a, b, (((1,), (0,)), ((), ())),
                           precision=lax.Precision.HIGHEST,
                           preferred_element_type=jnp.float32)


def _relu(x):
    return jnp.maximum(x, 0.0)


def _leaky(x):
    return jnp.maximum(x, 0.2 * x)


def _t0_body(f_ref, ws_ref, bs_ref, wn_ref, bn_ref, as_ref, an_ref, exp_ref,
             wu_ref, bu_ref, wp_ref, bp_ref, wf_ref, bf_ref, wsf_ref, bsf_ref,
             x5_ref, atse_ref, spre_ref):
    f = f_ref[...]
    fs = _relu(_mm(f, ws_ref[...]) + bs_ref[...])
    ats = _leaky(_mm(fs, as_ref[...]))
    atn = _leaky(_mm(fs, an_ref[...]))
    fn = _relu(_mm(f, wn_ref[...]) + bn_ref[...])
    x5_ref[0] = fn
    x5_ref[1] = _mm(atn, exp_ref[...]) * fn
    atse_ref[...] = _mm(ats, exp_ref[...])
    x5_ref[2] = _mm(f, wu_ref[...]) + bu_ref[...]
    x5_ref[3] = _mm(f, wp_ref[...]) + bp_ref[...]
    x5_ref[4] = _mm(f, wf_ref[...]) + bf_ref[...]
    spre_ref[...] = _relu(_mm(f, wsf_ref[...]) + bsf_ref[...])


def _t1_body(y5_ref, atse_ref, pg_ref, pu_ref, pp_ref, pf_ref, st_ref):
    ya = y5_ref[0, 0] + y5_ref[0, 1]
    yb = y5_ref[1, 0] + y5_ref[1, 1]
    pg = atse_ref[...] * ya + yb
    pu = _relu(y5_ref[2, 0] + y5_ref[2, 1])
    pp = _relu(y5_ref[3, 0] + y5_ref[3, 1])
    pf = _relu(y5_ref[4, 0] + y5_ref[4, 1])
    pg_ref[...] = pg
    pu_ref[...] = pu
    pp_ref[...] = pp
    pf_ref[...] = pf

    @pl.when(pl.program_id(0) == 0)
    def _():
        st_ref[...] = jnp.zeros_like(st_ref)

    st = jnp.concatenate([
        jnp.sum(pg, axis=0)[None], jnp.sum(pg * pg, axis=0)[None],
        jnp.sum(pu, axis=0)[None], jnp.sum(pu * pu, axis=0)[None],
        jnp.sum(pp, axis=0)[None], jnp.sum(pp * pp, axis=0)[None],
        jnp.sum(pf, axis=0)[None], jnp.sum(pf * pf, axis=0)[None],
    ], axis=0)
    st_ref[...] = st_ref[...] + st


def _bn_apply(p, s, ssq, gamma, beta):
    m = s / N
    v = ssq / N - m * m
    inv = lax.rsqrt(v + EPS)
    return (p - m) * inv * gamma + beta


def _t1post_body(pg_ref, pu_ref, pp_ref, pf_ref, st_ref,
                 gg_ref, gb_ref, ug_ref, ub_ref, pgm_ref, pbt_ref, fg_ref, fb_ref,
                 ws_ref, bs_ref, wn_ref, bn_ref, as_ref, an_ref, exp_ref,
                 wu_ref, bu_ref, wp_ref, bp_ref, wf_ref, bf_ref,
                 x5_ref, atse_ref):
    st = st_ref[...]
    fg = _bn_apply(pg_ref[...], st[0], st[1], gg_ref[...], gb_ref[...])
    fu = _bn_apply(pu_ref[...], st[2], st[3], ug_ref[...], ub_ref[...])
    fp = _bn_apply(pp_ref[...], st[4], st[5], pgm_ref[...], pbt_ref[...])
    ff = _bn_apply(pf_ref[...], st[6], st[7], fg_ref[...], fb_ref[...])

    fs = _relu(_mm(fg, ws_ref[...]) + bs_ref[...])
    ats = _leaky(_mm(fs, as_ref[...]))
    atn = _leaky(_mm(fs, an_ref[...]))
    fn = _relu(_mm(fg, wn_ref[...]) + bn_ref[...])
    x5_ref[0] = fn
    x5_ref[1] = _mm(atn, exp_ref[...]) * fn
    atse_ref[...] = _mm(ats, exp_ref[...])
    x5_ref[2] = _mm(fu, wu_ref[...]) + bu_ref[...]
    x5_ref[3] = _mm(fp, wp_ref[...]) + bp_ref[...]
    x5_ref[4] = _mm(ff, wf_ref[...]) + bf_ref[...]


def _t2_body(y5_ref, atse_ref, spre_ref,
             pg_ref, pu_ref, pp_ref, pf_ref, st_ref):
    ya = y5_ref[0, 0] + y5_ref[0, 1]
    yb = y5_ref[1, 0] + y5_ref[1, 1]
    pg = atse_ref[...] * ya + yb
    pu = _relu(y5_ref[2, 0] + y5_ref[2, 1])
    pp = _relu(y5_ref[3, 0] + y5_ref[3, 1])
    pf = _relu(y5_ref[4, 0] + y5_ref[4, 1])
    rowbase = pl.program_id(0) * BLK
    inb = (lax.broadcasted_iota(jnp.int32, (BLK, 1), 0) + rowbase) < N
    ps = jnp.where(inb, spre_ref[...], 0.0)
    pg_ref[...] = pg
    pu_ref[...] = pu
    pp_ref[...] = pp
    pf_ref[...] = pf

    @pl.when(pl.program_id(0) == 0)
    def _():
        st_ref[...] = jnp.zeros_like(st_ref)

    st = jnp.concatenate([
        jnp.sum(pg, axis=0)[None], jnp.sum(pg * pg, axis=0)[None],
        jnp.sum(pu, axis=0)[None], jnp.sum(pu * pu, axis=0)[None],
        jnp.sum(pp, axis=0)[None], jnp.sum(pp * pp, axis=0)[None],
        jnp.sum(pf, axis=0)[None], jnp.sum(pf * pf, axis=0)[None],
        jnp.sum(ps, axis=0)[None], jnp.sum(ps * ps, axis=0)[None],
    ], axis=0)
    st_ref[...] = st_ref[...] + st


def _t2post_body(pg_ref, pu_ref, pp_ref, pf_ref, ps_ref, st_ref,
                 gg_ref, gb_ref, ug_ref, ub_ref, pgm_ref, pbt_ref, fg_ref, fb_ref,
                 sg_ref, sb_ref,
                 wcg_ref, wcu_ref, wcp_ref, wcf_ref, wcs_ref, cb_ref, o_ref):
    st = st_ref[...]
    og = _bn_apply(pg_ref[...], st[0], st[1], gg_ref[...], gb_ref[...])
    ou = _bn_apply(pu_ref[...], st[2], st[3], ug_ref[...], ub_ref[...])
    op = _bn_apply(pp_ref[...], st[4], st[5], pgm_ref[...], pbt_ref[...])
    of = _bn_apply(pf_ref[...], st[6], st[7], fg_ref[...], fb_ref[...])
    os_ = _bn_apply(ps_ref[...], st[8], st[9], sg_ref[...], sb_ref[...])
    o_ref[...] = (_mm(og, wcg_ref[...]) + _mm(ou, wcu_ref[...])
                  + _mm(op, wcp_ref[...]) + _mm(of, wcf_ref[...])
                  + _mm(os_, wcs_ref[...]) + cb_ref[...])


def _sc_spmm(x5, rows5, cols5, vals5):
    mesh = plsc.VectorSubcoreMesh(core_axis_name="c", subcore_axis_name="s",
                                  num_cores=NC, num_subcores=NS)
    out_type = jax.ShapeDtypeStruct((5, NC, NPAD, D), jnp.float32)
    qtr = NCHW // 5
    scratch = [
        pltpu.VMEM_SHARED((NPAD, D), jnp.float32),
        pltpu.VMEM((qtr, CH), jnp.int32),
        pltpu.VMEM((qtr, CH), jnp.int32),
        pltpu.VMEM((qtr, CH), jnp.float32),
        pltpu.VMEM((CH, D), jnp.float32),
        pltpu.VMEM((CH, D), jnp.float32),
        pltpu.VMEM((64, D), jnp.float32),
        pltpu.SemaphoreType.DMA,
        pltpu.SemaphoreType.DMA,
        pltpu.SemaphoreType.DMA,
        pltpu.SemaphoreType.DMA,
    ]

    @functools.partial(pl.kernel, mesh=mesh, out_type=out_type,
                       scratch_types=scratch)
    def k(x_h, r_h, c_h, v_h, o_h, acc, rows_v, cols_v, vals_v, GA, GB, Z,
          sga, sgb, ssa, ssb):
        cid = lax.axis_index("c")
        sid = lax.axis_index("s")
        wid = sid * NC + cid

        @pl.loop(0, 64)
        def _(e):
            for r in range(D // 16):
                Z[e, pl.ds(r * 16, 16)] = jnp.zeros((16,), jnp.float32)

        def zero_slice():
            @pl.loop(0, RPW // 64)
            def _(kk):
                pltpu.sync_copy(Z, acc.at[pl.ds(sid * RPW + kk * 64, 64)])

        def scale(G, ci):
            for g in range((CH + 15) // 16):
                base = g * 16 if (g + 1) * 16 <= CH else CH - 16
                lo = 0 if (g + 1) * 16 <= CH else g * 16 - base
                vv = vals_v[ci, pl.ds(base, 16)]
                for j in range(lo, 16):
                    val = vv[j]
                    for r in range(D // 16):
                        sl = pl.ds(r * 16, 16)
                        G[base + j, sl] = G[base + j, sl] * val

        zero_slice()
        plsc.subcore_barrier()

        @pl.loop(0, 5)
        def _(p):
            @pl.loop(0, NCHW // qtr)
            def _(qp):
                qbase = wid * NCHW + qp * qtr
                pltpu.sync_copy(r_h.at[p, pl.ds(qbase, qtr)], rows_v)
                pltpu.sync_copy(c_h.at[p, pl.ds(qbase, qtr)], cols_v)
                pltpu.sync_copy(v_h.at[p, pl.ds(qbase, qtr)], vals_v)

                pltpu.async_copy(x_h.at[p].at[cols_v.at[0]], GA, sga)

                @pl.loop(0, qtr // 2)
                def _(kk):
                    ca = 2 * kk
                    cb = 2 * kk + 1
                    pltpu.make_async_copy(x_h.at[p].at[cols_v.at[ca]], GA,
                                          sga).wait()
                    pltpu.async_copy(x_h.at[p].at[cols_v.at[cb]], GB, sgb)
                    scale(GA, ca)
                    pltpu.sync_copy(GA, acc.at[rows_v.at[ca]], add=True)
                    pltpu.make_async_copy(x_h.at[p].at[cols_v.at[cb]], GB,
                                          sgb).wait()

                    @pl.when(cb + 1 < qtr)
                    def _():
                        pltpu.async_copy(x_h.at[p].at[cols_v.at[cb + 1]],
                                         GA, sga)

                    scale(GB, cb)
                    pltpu.sync_copy(GB, acc.at[rows_v.at[cb]], add=True)

            plsc.subcore_barrier()
            pltpu.sync_copy(acc.at[pl.ds(sid * RPW, RPW)],
                            o_h.at[p, cid, pl.ds(sid * RPW, RPW)])
            zero_slice()
            plsc.subcore_barrier()

    return k(x5, rows5, cols5, vals5)


def _blockdiag(a):
    return (a[:, :, None] * jnp.eye(H, dtype=a.dtype)[:, None, :]).reshape(D, H)


def _cat_w(w):
    return w.transpose(1, 0, 2).reshape(D, H * HD)


def kernel(f_in, edge_index0, adj_val0, edge_index1, adj_val1, edge_index2,
           adj_val2, edge_index3, adj_val3, gat_Wself, gat_bself, gat_Wneigh,
           gat_bneigh, gat_aself, gat_aneigh, gat_gamma, gat_beta, uttr_W,
           uttr_b, uttr_gamma, uttr_beta, past_W, past_b, past_gamma,
           past_beta, futr_W, futr_b, futr_gamma, futr_beta, self_W, self_b,
           self_gamma, self_beta, cls_W, cls_b):
    exp_mat = jnp.repeat(jnp.eye(H, dtype=jnp.float32), HD, axis=1)

    eis = [edge_index0, edge_index0, edge_index1, edge_index2, edge_index3]
    avs = [adj_val0, adj_val0, adj_val1, adj_val2, adj_val3]
    rows5 = jnp.stack([ei[0].reshape(E // CH, CH) for ei in eis])
    cols5 = jnp.stack([ei[1].reshape(E // CH, CH) for ei in eis])
    vals5 = jnp.stack([av.reshape(E // CH, CH) for av in avs])

    full = lambda s: pl.BlockSpec(s, lambda i: tuple(0 for _ in s))
    rblk = pl.BlockSpec((BLK, D), lambda i: (i, 0))
    x5blk = pl.BlockSpec((5, BLK, D), lambda i: (0, i, 0))
    y5blk = pl.BlockSpec((5, NC, BLK, D), lambda i: (0, 0, i, 0))
    w128 = full((D, D))
    b128 = full((D,))
    a8 = full((D, H))
    e8 = full((H, D))

    f_pad = jnp.pad(f_in, ((0, NPAD - N), (0, 0)))
    bd = jax.ShapeDtypeStruct((NPAD, D), jnp.float32)

    t0_out = pl.pallas_call(
        _t0_body,
        grid=(NBLK,),
        in_specs=[rblk] + [w128, b128, w128, b128, a8, a8, e8]
        + [w128, b128, w128, b128, w128, b128, w128, b128],
        out_specs=[x5blk, rblk, rblk],
        out_shape=[jax.ShapeDtypeStruct((5, NPAD, D), jnp.float32), bd, bd],
    )(f_pad, _cat_w(gat_Wself[0]), gat_bself[0].reshape(D),
      _cat_w(gat_Wneigh[0]), gat_bneigh[0].reshape(D),
      _blockdiag(gat_aself[0, :, :, 0]), _blockdiag(gat_aneigh[0, :, :, 0]),
      exp_mat, uttr_W[0], uttr_b[0], past_W[0], past_b[0],
      futr_W[0], futr_b[0], self_W, self_b)
    x5, atse0, spre = t0_out

    y5 = _sc_spmm(x5, rows5, cols5, vals5)

    pg1, pu1, pp1, pf1, st1 = pl.pallas_call(
        _t1_body,
        grid=(NBLK,),
        in_specs=[y5blk, rblk],
        out_specs=[rblk] * 4 + [full((8, D))],
        out_shape=[bd] * 4 + [jax.ShapeDtypeStruct((8, D), jnp.float32)],
    )(y5, atse0)

    t1p_out = pl.pallas_call(
        _t1post_body,
        grid=(NBLK,),
        in_specs=[rblk] * 4 + [full((8, D))] + [b128] * 8
        + [w128, b128, w128, b128, a8, a8, e8]
        + [w128, b128, w128, b128, w128, b128],
        out_specs=[x5blk, rblk],
        out_shape=[jax.ShapeDtypeStruct((5, NPAD, D), jnp.float32), bd],
    )(pg1, pu1, pp1, pf1, st1,
      gat_gamma[0], gat_beta[0], uttr_gamma[0], uttr_beta[0],
      past_gamma[0], past_beta[0], futr_gamma[0], futr_beta[0],
      _cat_w(gat_Wself[1]), gat_bself[1].reshape(D),
      _cat_w(gat_Wneigh[1]), gat_bneigh[1].reshape(D),
      _blockdiag(gat_aself[1, :, :, 0]), _blockdiag(gat_aneigh[1, :, :, 0]),
      exp_mat, uttr_W[1], uttr_b[1], past_W[1], past_b[1], futr_W[1], futr_b[1])
    x5_2, atse1 = t1p_out

    y5_2 = _sc_spmm(x5_2, rows5, cols5, vals5)

    pg2, pu2, pp2, pf2, st2 = pl.pallas_call(
        _t2_body,
        grid=(NBLK,),
        in_specs=[y5blk, rblk, rblk],
        out_specs=[rblk] * 4 + [full((10, D))],
        out_shape=[bd] * 4 + [jax.ShapeDtypeStruct((10, D), jnp.float32)],
    )(y5_2, atse1, spre)

    out = pl.pallas_call(
        _t2post_body,
        grid=(NBLK,),
        in_specs=[rblk] * 5 + [full((10, D))] + [b128] * 10
        + [full((D, C))] * 5 + [full((C,))],
        out_specs=pl.BlockSpec((BLK, C), lambda i: (i, 0)),
        out_shape=jax.ShapeDtypeStruct((NPAD, C), jnp.float32),
    )(pg2, pu2, pp2, pf2, spre, st2,
      gat_gamma[1], gat_beta[1], uttr_gamma[1], uttr_beta[1],
      past_gamma[1], past_beta[1], futr_gamma[1], futr_beta[1],
      self_gamma, self_beta,
      cls_W[0:D], cls_W[D:2 * D], cls_W[2 * D:3 * D], cls_W[3 * D:4 * D],
      cls_W[4 * D:5 * D], cls_b)
    return out[:N]
